# baseline (speedup 1.0000x reference)
import functools
import math

import jax
import jax.numpy as jnp
from jax import lax
from jax.experimental import pallas as pl
from jax.experimental.pallas import tpu as pltpu
from jax.experimental.pallas import tpu_sc as plsc

_EMB = 128
_SCALE = math.sqrt(_EMB)

_NC = 2
_NS = 16
_NW = _NC * _NS

_CHUNK = 128
_PAIR = 2 * _CHUNK


def _make_gather(n_idx):
    per_w = n_idx // _NW
    n_chunks = per_w // _CHUNK
    n_pairs = n_chunks // 2
    assert per_w * _NW == n_idx and n_pairs * _PAIR == per_w
    assert n_pairs % 2 == 0 and n_pairs >= 4
    mesh = plsc.VectorSubcoreMesh(core_axis_name="c", subcore_axis_name="s")

    @functools.partial(
        pl.kernel,
        mesh=mesh,
        out_type=jax.ShapeDtypeStruct((n_idx, _EMB), jnp.float32),
        scratch_types=[
            pltpu.VMEM((n_chunks, _CHUNK), jnp.int32),
            pltpu.VMEM((2 * _PAIR, _EMB), jnp.float32),
            pltpu.SemaphoreType.DMA,
            pltpu.SemaphoreType.DMA,
            pltpu.SemaphoreType.DMA,
            pltpu.SemaphoreType.DMA,
        ],
    )
    def k(table_hbm, idx_hbm, out_hbm, idx_v, rows_v, g0, g1, s0, s1):
        sem_g = (g0, g1)
        sem_s = (s0, s1)
        wid = lax.axis_index("s") * _NC + lax.axis_index("c")
        gbase = wid * n_chunks

        pltpu.sync_copy(idx_hbm.at[pl.ds(gbase, n_chunks)], idx_v)

        def gather_pair(p, buf):
            for h in range(2):
                pltpu.async_copy(
                    table_hbm.at[idx_v.at[2 * p + h]],
                    rows_v.at[pl.ds(buf * _PAIR + h * _CHUNK, _CHUNK)],
                    sem_g[buf])

        def wait_gather(buf):
            pltpu.make_async_copy(
                table_hbm.at[pl.ds(0, _PAIR)],
                rows_v.at[pl.ds(buf * _PAIR, _PAIR)], sem_g[buf]).wait()

        def scatter_pair(p, buf):
            pltpu.async_copy(
                rows_v.at[pl.ds(buf * _PAIR, _PAIR)],
                out_hbm.at[pl.ds(gbase * _CHUNK + p * _PAIR, _PAIR)],
                sem_s[buf])

        def wait_scatter(buf):
            pltpu.make_async_copy(
                rows_v.at[pl.ds(buf * _PAIR, _PAIR)],
                out_hbm.at[pl.ds(0, _PAIR)], sem_s[buf]).wait()

        def scale(buf):
            @plsc.parallel_loop(0, _PAIR, step=2)
            def _(r):
                base = buf * _PAIR
                for rr in range(2):
                    for j in range(_EMB // 16):
                        sl = pl.ds(j * 16, 16)
                        rows_v[base + r + rr, sl] = (
                            rows_v[base + r + rr, sl] * _SCALE)

        gather_pair(0, 0)
        gather_pair(1, 1)
        wait_gather(0)
        scale(0)
        scatter_pair(0, 0)

        def body(o, carry):
            for b in range(2):
                p = 2 * o + b
                wait_scatter(b)
                gather_pair(p, b)
                wait_gather(1 - b)
                scale(1 - b)
                scatter_pair(p - 1, 1 - b)
            return carry

        lax.fori_loop(1, n_pairs // 2, body, 0, unroll=False)

        wait_gather(1)
        scale(1)
        scatter_pair(n_pairs - 1, 1)
        wait_scatter(0)
        wait_scatter(1)

    return k


def kernel(tokens, table):
    b, l = tokens.shape
    n_idx = b * l
    idx = tokens.reshape(n_idx // _CHUNK, _CHUNK)
    out = _make_gather(n_idx)(table, idx)
    return out.reshape(b, l, _EMB)


# speedup vs baseline: 1.0012x; 1.0012x over previous
import functools
import math

import jax
import jax.numpy as jnp
from jax import lax
from jax.experimental import pallas as pl
from jax.experimental.pallas import tpu as pltpu
from jax.experimental.pallas import tpu_sc as plsc

_EMB = 128
_SCALE = math.sqrt(_EMB)

_NC = 2
_NS = 16
_NW = _NC * _NS

_CHUNK = 128
_NBUF = 4
_LAG = 2


def _make_gather(n_idx):
    per_w = n_idx // _NW
    n_chunks = per_w // _CHUNK
    assert per_w * _NW == n_idx and n_chunks * _CHUNK == per_w
    assert n_chunks % _NBUF == 0 and n_chunks >= 2 * _NBUF
    assert 2 * _LAG <= _NBUF
    mesh = plsc.VectorSubcoreMesh(core_axis_name="c", subcore_axis_name="s")

    @functools.partial(
        pl.kernel,
        mesh=mesh,
        out_type=jax.ShapeDtypeStruct((n_idx, _EMB), jnp.float32),
        scratch_types=[
            pltpu.VMEM((n_chunks, _CHUNK), jnp.int32),
            pltpu.VMEM((_NBUF, _CHUNK, _EMB), jnp.float32),
        ]
        + [pltpu.SemaphoreType.DMA] * (2 * _NBUF),
    )
    def k(table_hbm, idx_hbm, out_hbm, idx_v, rows_v, *sems):
        sem_g = sems[:_NBUF]
        sem_s = sems[_NBUF:]
        wid = lax.axis_index("s") * _NC + lax.axis_index("c")
        gbase = wid * n_chunks

        pltpu.sync_copy(idx_hbm.at[pl.ds(gbase, n_chunks)], idx_v)

        def gather(t, slot):
            pltpu.async_copy(
                table_hbm.at[idx_v.at[t]], rows_v.at[slot], sem_g[slot])

        def wait_gather(slot):
            pltpu.make_async_copy(
                table_hbm.at[pl.ds(0, _CHUNK)], rows_v.at[slot],
                sem_g[slot]).wait()

        def scatter(t, slot):
            pltpu.async_copy(
                rows_v.at[slot],
                out_hbm.at[pl.ds((gbase + t) * _CHUNK, _CHUNK)], sem_s[slot])

        def wait_scatter(slot):
            pltpu.make_async_copy(
                rows_v.at[slot], out_hbm.at[pl.ds(0, _CHUNK)],
                sem_s[slot]).wait()

        def scale(slot):
            @plsc.parallel_loop(0, _CHUNK, step=2)
            def _(r):
                for rr in range(2):
                    for j in range(_EMB // 16):
                        sl = pl.ds(j * 16, 16)
                        rows_v[slot, r + rr, sl] = (
                            rows_v[slot, r + rr, sl] * _SCALE)

        for u in range(_LAG):
            gather(u, u)
        for b in range(_NBUF - _LAG):
            u, s = _LAG + b, b
            gather(u, u)
            wait_gather(s)
            scale(s)
            scatter(s, s)
        for b in range(_LAG):
            u, s = _NBUF + b, _NBUF - _LAG + b
            wait_scatter(u % _NBUF)
            gather(u, u % _NBUF)
            wait_gather(s)
            scale(s)
            scatter(s, s)

        def body(o, carry):
            for b in range(_NBUF):
                u = _NBUF * o + _LAG + b
                s = _NBUF * o + b
                us = (_LAG + b) % _NBUF
                wait_scatter(us)
                gather(u, us)
                wait_gather(b)
                scale(b)
                scatter(s, b)
            return carry

        lax.fori_loop(1, n_chunks // _NBUF - 1, body, 0, unroll=False)

        for b in range(_NBUF - _LAG):
            u = n_chunks - (_NBUF - _LAG) + b
            wait_scatter(u % _NBUF)
            gather(u, u % _NBUF)
        for b in range(_NBUF):
            s = n_chunks - _NBUF + b
            wait_gather(s % _NBUF)
            scale(s % _NBUF)
            scatter(s, s % _NBUF)
        for b in range(_NBUF):
            wait_scatter(b)

    return k


def kernel(tokens, table):
    b, l = tokens.shape
    n_idx = b * l
    idx = tokens.reshape(n_idx // _CHUNK, _CHUNK)
    out = _make_gather(n_idx)(table, idx)
    return out.reshape(b, l, _EMB)
